# baseline (speedup 1.0000x reference)
import functools

import jax
import jax.numpy as jnp
from jax import lax
from jax.experimental import pallas as pl
from jax.experimental.pallas import tpu as pltpu
from jax.experimental.pallas import tpu_sc as plsc

B, S, D = 2, 2048, 1024
E, K = 64, 2
DFF = 2048
T = B * S
TK = T * K
C = int(2.0 * TK / E)
EC = E * C
TRASH = EC
TAB = EC + 16

NC, NS = 2, 16
NW = NC * NS
ROWS_W = EC // NW
TOK_W = T // NW


def _router_body(flat_ref, rw_ref, rb_ref, dest_ref, comb_ref, g_ref):
    flat = flat_ref[...]
    logits = jnp.dot(flat, rw_ref[...], preferred_element_type=jnp.float32)
    logits = logits + rb_ref[...]
    lane = lax.broadcasted_iota(jnp.int32, (T, E), 1)
    v0 = jnp.max(logits, axis=1, keepdims=True)
    i0 = jnp.min(jnp.where(logits == v0, lane, E), axis=1, keepdims=True)
    l2 = jnp.where(lane == i0, -jnp.inf, logits)
    v1 = jnp.max(l2, axis=1, keepdims=True)
    i1 = jnp.min(jnp.where(l2 == v1, lane, E), axis=1, keepdims=True)
    e1 = jnp.exp(v1 - v0)
    denom = 1.0 + e1
    g_ref[:, 0:1] = 1.0 / denom
    g_ref[:, 1:2] = e1 / denom
    oh0 = (lane == i0).astype(jnp.float32)
    oh1 = (lane == i1).astype(jnp.float32)
    ssum = oh0 + oh1
    BLK = 256
    ri = lax.broadcasted_iota(jnp.int32, (BLK, BLK), 0)
    ci = lax.broadcasted_iota(jnp.int32, (BLK, BLK), 1)
    tri = (ci < ri).astype(jnp.float32)
    carry = jnp.zeros((1, E), jnp.float32)
    p0, p1 = [], []
    for b in range(T // BLK):
        blk = ssum[b * BLK:(b + 1) * BLK, :]
        excl = jnp.dot(tri, blk, preferred_element_type=jnp.float32) + carry
        p0.append(jnp.sum(excl * oh0[b * BLK:(b + 1) * BLK, :], axis=1, keepdims=True))
        p1.append(jnp.sum(excl * oh1[b * BLK:(b + 1) * BLK, :], axis=1, keepdims=True))
        carry = carry + jnp.sum(blk, axis=0, keepdims=True)
    pos0 = jnp.concatenate(p0, axis=0).astype(jnp.int32)
    pos1 = jnp.concatenate(p1, axis=0).astype(jnp.int32)
    cmin = jnp.min(carry)
    lane1 = lax.broadcasted_iota(jnp.int32, (1, E), 1)
    emin = jnp.min(jnp.where(carry == cmin, lane1, E))
    zrow = emin * C + (C - 1)
    slot0 = i0 * C + pos0
    slot1 = i1 * C + pos1
    keep0 = pos0 < C
    keep1 = pos1 < C
    dest_ref[:, 0:1] = jnp.where(keep0, slot0, TRASH)
    dest_ref[:, 1:2] = jnp.where(keep1, slot1, TRASH)
    comb_ref[:, 0:1] = jnp.where(keep0, slot0, zrow)
    comb_ref[:, 1:2] = jnp.where(keep1, slot1, zrow)


def _router_call(flat, router_w, router_b):
    return pl.pallas_call(
        _router_body,
        out_shape=[
            jax.ShapeDtypeStruct((T, K), jnp.int32),
            jax.ShapeDtypeStruct((T, K), jnp.int32),
            jax.ShapeDtypeStruct((T, K), jnp.float32),
        ],
    )(flat, router_w, router_b)


def _dispatch_body(flat_hbm, dest_hbm, gsc_hbm, buf_hbm, gw_hbm,
                   didx, gscv, src_tab, gw_tab, rows, sem):
    wid = lax.axis_index("s") * NC + lax.axis_index("c")
    pltpu.sync_copy(dest_hbm, didx)
    pltpu.sync_copy(gsc_hbm, gscv)
    zi = jnp.zeros((16,), jnp.int32)
    zf = jnp.zeros((16,), jnp.float32)

    def zero_body(i, _):
        src_tab[pl.ds(i * 16, 16)] = zi
        gw_tab[pl.ds(i * 16, 16)] = zf
        return 0

    lax.fori_loop(0, TAB // 16, zero_body, 0)
    lane = lax.broadcasted_iota(jnp.int32, (16,), 0)

    def scat_body(i, _):
        o = i * 16
        idx = didx[pl.ds(o, 16)]
        tok = lax.shift_right_logical(o + lane, 1)
        plsc.store_scatter(src_tab, [idx], tok)
        plsc.store_scatter(gw_tab, [idx], gscv[pl.ds(o, 16)])
        return 0

    lax.fori_loop(0, TK // 16, scat_body, 0)
    base = wid * ROWS_W
    CH = 32

    def gath_body(i, _):
        off = base + i * CH
        pltpu.async_copy(flat_hbm.at[src_tab.at[pl.ds(off, CH)]], rows, sem).wait()
        pltpu.sync_copy(rows, buf_hbm.at[pl.ds(off, CH)])
        return 0

    lax.fori_loop(0, ROWS_W // CH, gath_body, 0)
    pltpu.sync_copy(gw_tab.at[pl.ds(base, ROWS_W)], gw_hbm.at[pl.ds(base, ROWS_W)])


@functools.lru_cache(maxsize=None)
def _dispatch_kernel():
    return pl.kernel(
        _dispatch_body,
        out_type=[
            jax.ShapeDtypeStruct((EC, D), jnp.float32),
            jax.ShapeDtypeStruct((EC,), jnp.float32),
        ],
        mesh=plsc.VectorSubcoreMesh(core_axis_name="c", subcore_axis_name="s",
                                    num_cores=NC, num_subcores=NS),
        compiler_params=pltpu.CompilerParams(needs_layout_passes=False),
        scratch_types=[
            pltpu.VMEM((TK,), jnp.int32),
            pltpu.VMEM((TK,), jnp.float32),
            pltpu.VMEM((TAB,), jnp.int32),
            pltpu.VMEM((TAB,), jnp.float32),
            pltpu.VMEM((32, D), jnp.float32),
            pltpu.SemaphoreType.DMA,
        ],
    )


def _ffn_body(buf_ref, w1_ref, w2_ref, gw_ref, yw_ref):
    xb = buf_ref[0]
    h = jax.nn.gelu(jnp.dot(xb, w1_ref[0], preferred_element_type=jnp.float32))
    y = jnp.dot(h, w2_ref[0], preferred_element_type=jnp.float32)
    yw_ref[0] = y * gw_ref[0]


def _ffn_call(buf3, w1, w2, gw3):
    return pl.pallas_call(
        _ffn_body,
        grid=(E,),
        in_specs=[
            pl.BlockSpec((1, C, D), lambda e: (e, 0, 0)),
            pl.BlockSpec((1, D, DFF), lambda e: (e, 0, 0)),
            pl.BlockSpec((1, DFF, D), lambda e: (e, 0, 0)),
            pl.BlockSpec((1, C, 1), lambda e: (e, 0, 0)),
        ],
        out_specs=pl.BlockSpec((1, C, D), lambda e: (e, 0, 0)),
        out_shape=jax.ShapeDtypeStruct((E, C, D), jnp.float32),
    )(buf3, w1, w2, gw3)


def _combine_body(yw_hbm, comb_hbm, out_hbm, cidx, rows, outv, sem):
    wid = lax.axis_index("s") * NC + lax.axis_index("c")
    tbase = wid * TOK_W
    pltpu.sync_copy(comb_hbm.at[pl.ds(tbase * K, TOK_W * K)], cidx)
    CT = 16

    def chunk_body(ci, _):
        pltpu.async_copy(yw_hbm.at[cidx.at[pl.ds(ci * CT * K, CT * K)]], rows, sem).wait()

        def row_body(r, _2):
            for q in range(D // 16):
                sl = pl.ds(q * 16, 16)
                outv[r, sl] = rows[2 * r, sl] + rows[2 * r + 1, sl]
            return 0

        lax.fori_loop(0, CT, row_body, 0)
        pltpu.sync_copy(outv, out_hbm.at[pl.ds(tbase + ci * CT, CT)])
        return 0

    lax.fori_loop(0, TOK_W // CT, chunk_body, 0)


@functools.lru_cache(maxsize=None)
def _combine_kernel():
    return pl.kernel(
        _combine_body,
        out_type=jax.ShapeDtypeStruct((T, D), jnp.float32),
        mesh=plsc.VectorSubcoreMesh(core_axis_name="c", subcore_axis_name="s",
                                    num_cores=NC, num_subcores=NS),
        compiler_params=pltpu.CompilerParams(needs_layout_passes=False),
        scratch_types=[
            pltpu.VMEM((TOK_W * K,), jnp.int32),
            pltpu.VMEM((2 * 16, D), jnp.float32),
            pltpu.VMEM((16, D), jnp.float32),
            pltpu.SemaphoreType.DMA,
        ],
    )


def kernel(hidden_states, router_w, router_b, w1, w2):
    flat = hidden_states.reshape(T, D)
    dest, comb, g = _router_call(flat, router_w, router_b.reshape(1, E))
    buf, gw = _dispatch_kernel()(flat, dest.reshape(TK), g.reshape(TK))
    yw = _ffn_call(buf.reshape(E, C, D), w1, w2, gw.reshape(E, C, 1))
    out = _combine_kernel()(yw.reshape(EC, D), comb.reshape(TK))
    return out.reshape(B, S, D)


# speedup vs baseline: 1.7833x; 1.7833x over previous
import functools

import jax
import jax.numpy as jnp
from jax import lax
from jax.experimental import pallas as pl
from jax.experimental.pallas import tpu as pltpu
from jax.experimental.pallas import tpu_sc as plsc

B, S, D = 2, 2048, 1024
E, K = 64, 2
DFF = 2048
T = B * S
TK = T * K
C = int(2.0 * TK / E)
EC = E * C

NC, NS = 2, 16
NW = NC * NS
TOK_W = T // NW
TCH = 32
NCH = TOK_W // TCH


def _router_body(flat_ref, rw_ref, rb_ref, comb_ref, g_ref, mask_ref):
    flat = flat_ref[...]
    logits = jnp.dot(flat, rw_ref[...], preferred_element_type=jnp.float32)
    logits = logits + rb_ref[...]
    lane = lax.broadcasted_iota(jnp.int32, (T, E), 1)
    v0 = jnp.max(logits, axis=1, keepdims=True)
    i0 = jnp.min(jnp.where(logits == v0, lane, E), axis=1, keepdims=True)
    l2 = jnp.where(lane == i0, -jnp.inf, logits)
    v1 = jnp.max(l2, axis=1, keepdims=True)
    i1 = jnp.min(jnp.where(l2 == v1, lane, E), axis=1, keepdims=True)
    e1 = jnp.exp(v1 - v0)
    denom = 1.0 + e1
    g_ref[:, 0:1] = 1.0 / denom
    g_ref[:, 1:2] = e1 / denom
    oh0 = (lane == i0).astype(jnp.float32)
    oh1 = (lane == i1).astype(jnp.float32)
    ssum = oh0 + oh1
    BLK = 256
    ri = lax.broadcasted_iota(jnp.int32, (BLK, BLK), 0)
    ci = lax.broadcasted_iota(jnp.int32, (BLK, BLK), 1)
    tri = (ci < ri).astype(jnp.float32)
    carry = jnp.zeros((1, E), jnp.float32)
    p0, p1 = [], []
    for b in range(T // BLK):
        blk = ssum[b * BLK:(b + 1) * BLK, :]
        excl = jnp.dot(tri, blk, preferred_element_type=jnp.float32) + carry
        p0.append(jnp.sum(excl * oh0[b * BLK:(b + 1) * BLK, :], axis=1, keepdims=True))
        p1.append(jnp.sum(excl * oh1[b * BLK:(b + 1) * BLK, :], axis=1, keepdims=True))
        carry = carry + jnp.sum(blk, axis=0, keepdims=True)
    pos0 = jnp.concatenate(p0, axis=0).astype(jnp.int32)
    pos1 = jnp.concatenate(p1, axis=0).astype(jnp.int32)
    cmin = jnp.min(carry)
    lane1 = lax.broadcasted_iota(jnp.int32, (1, E), 1)
    emin = jnp.min(jnp.where(carry == cmin, lane1, E))
    zrow = emin * C + (C - 1)
    slot0 = i0 * C + pos0
    slot1 = i1 * C + pos1
    comb_ref[:, 0:1] = jnp.where(pos0 < C, slot0, zrow)
    comb_ref[:, 1:2] = jnp.where(pos1 < C, slot1, zrow)
    ones = jnp.ones((T, 1), jnp.float32)
    cnt_col = lax.dot_general(ssum, ones, (((0,), (0,)), ((), ())),
                              preferred_element_type=jnp.float32)
    slot_iota = lax.broadcasted_iota(jnp.int32, (E, C), 1).astype(jnp.float32)
    mask_ref[...] = (slot_iota < cnt_col).astype(jnp.float32)


def _router_call(flat, router_w, router_b):
    return pl.pallas_call(
        _router_body,
        out_shape=[
            jax.ShapeDtypeStruct((T, K), jnp.int32),
            jax.ShapeDtypeStruct((T, K), jnp.float32),
            jax.ShapeDtypeStruct((E, C), jnp.float32),
        ],
    )(flat, router_w, router_b)


def _dispatch_body(flat_hbm, comb_hbm, gsc_hbm, buf_hbm, gw_hbm,
                   cfull, gfull, gw_tab, rows, idx0, idx1, semg, sems):
    wid = lax.axis_index("s") * NC + lax.axis_index("c")
    tbase = wid * TOK_W
    pltpu.sync_copy(comb_hbm, cfull)
    pltpu.sync_copy(gsc_hbm, gfull)
    gets = [None] * NCH
    for c in range(2):
        gets[c] = pltpu.async_copy(
            flat_hbm.at[pl.ds(tbase + c * TCH, TCH)], rows[c % 2], semg[c % 2])

    def scat_body(i, _):
        o = i * 16
        idx = cfull[pl.ds(o, 16)]
        plsc.store_scatter(gw_tab, [idx], gfull[pl.ds(o, 16)])
        return 0

    lax.fori_loop(0, TK // 16, scat_body, 0)

    lane = lax.broadcasted_iota(jnp.int32, (16,), 0)
    puts = [None] * NCH
    for c in range(NCH):
        p = c % 2
        jb = (tbase + c * TCH) * K
        for h in range(TCH // 16):
            idx0[p][pl.ds(h * 16, 16)] = plsc.load_gather(
                cfull, [jb + 2 * (h * 16 + lane)])
            idx1[p][pl.ds(h * 16, 16)] = plsc.load_gather(
                cfull, [jb + 2 * (h * 16 + lane) + 1])
        gets[c].wait()
        puts[c] = (
            pltpu.async_copy(rows[p], buf_hbm.at[idx0[p]], sems[p]),
            pltpu.async_copy(rows[p], buf_hbm.at[idx1[p]], sems[p]),
        )
        if c + 2 < NCH:
            puts[c][0].wait()
            puts[c][1].wait()
            puts[c] = None
            gets[c + 2] = pltpu.async_copy(
                flat_hbm.at[pl.ds(tbase + (c + 2) * TCH, TCH)], rows[p], semg[p])
    for c in range(NCH):
        if puts[c] is not None:
            puts[c][0].wait()
            puts[c][1].wait()
    pltpu.sync_copy(gw_tab.at[pl.ds(wid * (EC // NW), EC // NW)],
                    gw_hbm.at[pl.ds(wid * (EC // NW), EC // NW)])


@functools.lru_cache(maxsize=None)
def _dispatch_kernel():
    return pl.kernel(
        _dispatch_body,
        out_type=[
            jax.ShapeDtypeStruct((EC, D), jnp.float32),
            jax.ShapeDtypeStruct((EC,), jnp.float32),
        ],
        mesh=plsc.VectorSubcoreMesh(core_axis_name="c", subcore_axis_name="s",
                                    num_cores=NC, num_subcores=NS),
        compiler_params=pltpu.CompilerParams(needs_layout_passes=False),
        scratch_types=[
            pltpu.VMEM((TK,), jnp.int32),
            pltpu.VMEM((TK,), jnp.float32),
            pltpu.VMEM((EC,), jnp.float32),
            [pltpu.VMEM((TCH, D), jnp.float32)] * 2,
            [pltpu.VMEM((TCH,), jnp.int32)] * 2,
            [pltpu.VMEM((TCH,), jnp.int32)] * 2,
            [pltpu.SemaphoreType.DMA] * 2,
            [pltpu.SemaphoreType.DMA] * 2,
        ],
    )


def _ffn_body(buf_ref, w1_ref, w2_ref, gw_ref, m_ref, yw_ref):
    xb = buf_ref[0]
    h = jax.nn.gelu(jnp.dot(xb, w1_ref[0], preferred_element_type=jnp.float32))
    y = jnp.dot(h, w2_ref[0], preferred_element_type=jnp.float32)
    yw_ref[0] = jnp.where(m_ref[0] > 0, y * gw_ref[0], 0.0)


def _ffn_call(buf3, w1, w2, gw3, m3):
    return pl.pallas_call(
        _ffn_body,
        grid=(E,),
        in_specs=[
            pl.BlockSpec((1, C, D), lambda e: (e, 0, 0)),
            pl.BlockSpec((1, D, DFF), lambda e: (e, 0, 0)),
            pl.BlockSpec((1, DFF, D), lambda e: (e, 0, 0)),
            pl.BlockSpec((1, C, 1), lambda e: (e, 0, 0)),
            pl.BlockSpec((1, C, 1), lambda e: (e, 0, 0)),
        ],
        out_specs=pl.BlockSpec((1, C, D), lambda e: (e, 0, 0)),
        out_shape=jax.ShapeDtypeStruct((E, C, D), jnp.float32),
    )(buf3, w1, w2, gw3, m3)


def _combine_body(yw_hbm, comb_hbm, out_hbm, cidx, rows, outv, sem):
    wid = lax.axis_index("s") * NC + lax.axis_index("c")
    tbase = wid * TOK_W
    pltpu.sync_copy(comb_hbm.at[pl.ds(tbase * K, TOK_W * K)], cidx)
    CT = 16

    def chunk_body(ci, _):
        pltpu.async_copy(yw_hbm.at[cidx.at[pl.ds(ci * CT * K, CT * K)]], rows, sem).wait()

        def row_body(r, _2):
            for q in range(D // 16):
                sl = pl.ds(q * 16, 16)
                outv[r, sl] = rows[2 * r, sl] + rows[2 * r + 1, sl]
            return 0

        lax.fori_loop(0, CT, row_body, 0)
        pltpu.sync_copy(outv, out_hbm.at[pl.ds(tbase + ci * CT, CT)])
        return 0

    lax.fori_loop(0, TOK_W // CT, chunk_body, 0)


@functools.lru_cache(maxsize=None)
def _combine_kernel():
    return pl.kernel(
        _combine_body,
        out_type=jax.ShapeDtypeStruct((T, D), jnp.float32),
        mesh=plsc.VectorSubcoreMesh(core_axis_name="c", subcore_axis_name="s",
                                    num_cores=NC, num_subcores=NS),
        compiler_params=pltpu.CompilerParams(needs_layout_passes=False),
        scratch_types=[
            pltpu.VMEM((TOK_W * K,), jnp.int32),
            pltpu.VMEM((2 * 16, D), jnp.float32),
            pltpu.VMEM((16, D), jnp.float32),
            pltpu.SemaphoreType.DMA,
        ],
    )


def kernel(hidden_states, router_w, router_b, w1, w2):
    flat = hidden_states.reshape(T, D)
    comb, g, rowmask = _router_call(flat, router_w, router_b.reshape(1, E))
    buf, gw = _dispatch_kernel()(flat, comb.reshape(TK), g.reshape(TK))
    yw = _ffn_call(buf.reshape(E, C, D), w1, w2, gw.reshape(E, C, 1),
                   rowmask.reshape(E, C, 1))
    out = _combine_kernel()(yw.reshape(EC, D), comb.reshape(TK))
    return out.reshape(B, S, D)


# speedup vs baseline: 1.8517x; 1.0383x over previous
import functools

import jax
import jax.numpy as jnp
from jax import lax
from jax.experimental import pallas as pl
from jax.experimental.pallas import tpu as pltpu
from jax.experimental.pallas import tpu_sc as plsc

B, S, D = 2, 2048, 1024
E, K = 64, 2
DFF = 2048
T = B * S
TK = T * K
C = int(2.0 * TK / E)
EC = E * C

NC, NS = 2, 16
NW = NC * NS
TOK_W = T // NW
TCH = 32
NCH = TOK_W // TCH


def _router_body(flat_ref, rw_ref, rb_ref, comb_ref, g_ref, mask_ref):
    flat = flat_ref[...]
    logits = jnp.dot(flat, rw_ref[...], preferred_element_type=jnp.float32)
    logits = logits + rb_ref[...]
    lane = lax.broadcasted_iota(jnp.int32, (T, E), 1)
    v0 = jnp.max(logits, axis=1, keepdims=True)
    i0 = jnp.min(jnp.where(logits == v0, lane, E), axis=1, keepdims=True)
    l2 = jnp.where(lane == i0, -jnp.inf, logits)
    v1 = jnp.max(l2, axis=1, keepdims=True)
    i1 = jnp.min(jnp.where(l2 == v1, lane, E), axis=1, keepdims=True)
    e1 = jnp.exp(v1 - v0)
    denom = 1.0 + e1
    g_ref[:, 0:1] = 1.0 / denom
    g_ref[:, 1:2] = e1 / denom
    oh0 = (lane == i0).astype(jnp.float32)
    oh1 = (lane == i1).astype(jnp.float32)
    ssum = oh0 + oh1
    BLK = 256
    ri = lax.broadcasted_iota(jnp.int32, (BLK, BLK), 0)
    ci = lax.broadcasted_iota(jnp.int32, (BLK, BLK), 1)
    tri = (ci < ri).astype(jnp.float32)
    carry = jnp.zeros((1, E), jnp.float32)
    p0, p1 = [], []
    for b in range(T // BLK):
        blk = ssum[b * BLK:(b + 1) * BLK, :]
        excl = jnp.dot(tri, blk, preferred_element_type=jnp.float32) + carry
        p0.append(jnp.sum(excl * oh0[b * BLK:(b + 1) * BLK, :], axis=1, keepdims=True))
        p1.append(jnp.sum(excl * oh1[b * BLK:(b + 1) * BLK, :], axis=1, keepdims=True))
        carry = carry + jnp.sum(blk, axis=0, keepdims=True)
    pos0 = jnp.concatenate(p0, axis=0).astype(jnp.int32)
    pos1 = jnp.concatenate(p1, axis=0).astype(jnp.int32)
    cmin = jnp.min(carry)
    lane1 = lax.broadcasted_iota(jnp.int32, (1, E), 1)
    emin = jnp.min(jnp.where(carry == cmin, lane1, E))
    zrow = emin * C + (C - 1)
    slot0 = i0 * C + pos0
    slot1 = i1 * C + pos1
    comb_ref[:, 0:1] = jnp.where(pos0 < C, slot0, zrow)
    comb_ref[:, 1:2] = jnp.where(pos1 < C, slot1, zrow)
    ones = jnp.ones((T, 1), jnp.float32)
    cnt_col = lax.dot_general(ssum, ones, (((0,), (0,)), ((), ())),
                              preferred_element_type=jnp.float32)
    slot_iota = lax.broadcasted_iota(jnp.int32, (E, C), 1).astype(jnp.float32)
    mask_ref[...] = (slot_iota < cnt_col).astype(jnp.float32)


def _router_call(flat, router_w, router_b):
    return pl.pallas_call(
        _router_body,
        out_shape=[
            jax.ShapeDtypeStruct((T, K), jnp.int32),
            jax.ShapeDtypeStruct((T, K), jnp.float32),
            jax.ShapeDtypeStruct((E, C), jnp.float32),
        ],
    )(flat, router_w, router_b)


def _dispatch_body(flat_hbm, comb_hbm, gsc_hbm, buf_hbm, gw_hbm,
                   cfull, gfull, gw_tab, rows, idx0, idx1, semg, sems):
    wid = lax.axis_index("s") * NC + lax.axis_index("c")
    tbase = wid * TOK_W
    pltpu.sync_copy(comb_hbm, cfull)
    pltpu.sync_copy(gsc_hbm, gfull)
    gets = [None] * NCH
    for c in range(2):
        gets[c] = pltpu.async_copy(
            flat_hbm.at[pl.ds(tbase + c * TCH, TCH)], rows[c % 2], semg[c % 2])

    def scat_body(i, _):
        o = i * 16
        idx = cfull[pl.ds(o, 16)]
        plsc.store_scatter(gw_tab, [idx], gfull[pl.ds(o, 16)])
        return 0

    lax.fori_loop(0, TK // 16, scat_body, 0)

    lane = lax.broadcasted_iota(jnp.int32, (16,), 0)
    puts = [None] * NCH
    for c in range(NCH):
        p = c % 2
        jb = (tbase + c * TCH) * K
        for h in range(TCH // 16):
            idx0[p][pl.ds(h * 16, 16)] = plsc.load_gather(
                cfull, [jb + 2 * (h * 16 + lane)])
            idx1[p][pl.ds(h * 16, 16)] = plsc.load_gather(
                cfull, [jb + 2 * (h * 16 + lane) + 1])
        gets[c].wait()
        puts[c] = (
            pltpu.async_copy(rows[p], buf_hbm.at[idx0[p]], sems[p]),
            pltpu.async_copy(rows[p], buf_hbm.at[idx1[p]], sems[p]),
        )
        if c + 2 < NCH:
            puts[c][0].wait()
            puts[c][1].wait()
            puts[c] = None
            gets[c + 2] = pltpu.async_copy(
                flat_hbm.at[pl.ds(tbase + (c + 2) * TCH, TCH)], rows[p], semg[p])
    for c in range(NCH):
        if puts[c] is not None:
            puts[c][0].wait()
            puts[c][1].wait()
    pltpu.sync_copy(gw_tab.at[pl.ds(wid * (EC // NW), EC // NW)],
                    gw_hbm.at[pl.ds(wid * (EC // NW), EC // NW)])


@functools.lru_cache(maxsize=None)
def _dispatch_kernel():
    return pl.kernel(
        _dispatch_body,
        out_type=[
            jax.ShapeDtypeStruct((EC, D), jnp.float32),
            jax.ShapeDtypeStruct((EC,), jnp.float32),
        ],
        mesh=plsc.VectorSubcoreMesh(core_axis_name="c", subcore_axis_name="s",
                                    num_cores=NC, num_subcores=NS),
        compiler_params=pltpu.CompilerParams(needs_layout_passes=False),
        scratch_types=[
            pltpu.VMEM((TK,), jnp.int32),
            pltpu.VMEM((TK,), jnp.float32),
            pltpu.VMEM((EC,), jnp.float32),
            [pltpu.VMEM((TCH, D), jnp.float32)] * 2,
            [pltpu.VMEM((TCH,), jnp.int32)] * 2,
            [pltpu.VMEM((TCH,), jnp.int32)] * 2,
            [pltpu.SemaphoreType.DMA] * 2,
            [pltpu.SemaphoreType.DMA] * 2,
        ],
    )


def _ffn_body(buf_ref, w1_ref, w2_ref, gw_ref, m_ref, yw_ref):
    xb = buf_ref[0]
    h = jax.nn.gelu(jnp.dot(xb, w1_ref[0], preferred_element_type=jnp.float32))
    y = jnp.dot(h, w2_ref[0], preferred_element_type=jnp.float32)
    yw_ref[0] = jnp.where(m_ref[0] > 0, y * gw_ref[0], 0.0)


def _ffn_call(buf3, w1, w2, gw3, m3):
    return pl.pallas_call(
        _ffn_body,
        grid=(E,),
        in_specs=[
            pl.BlockSpec((1, C, D), lambda e: (e, 0, 0)),
            pl.BlockSpec((1, D, DFF), lambda e: (e, 0, 0)),
            pl.BlockSpec((1, DFF, D), lambda e: (e, 0, 0)),
            pl.BlockSpec((1, C, 1), lambda e: (e, 0, 0)),
            pl.BlockSpec((1, C, 1), lambda e: (e, 0, 0)),
        ],
        out_specs=pl.BlockSpec((1, C, D), lambda e: (e, 0, 0)),
        out_shape=jax.ShapeDtypeStruct((E, C, D), jnp.float32),
    )(buf3, w1, w2, gw3, m3)


CT = 16
NCC = TOK_W // CT


def _combine_body(yw_hbm, comb_hbm, out_hbm, cidx, rows, outv, semg, semp):
    wid = lax.axis_index("s") * NC + lax.axis_index("c")
    tbase = wid * TOK_W
    pltpu.sync_copy(comb_hbm.at[pl.ds(tbase * K, TOK_W * K)], cidx)
    gets = [None] * NCC
    puts = [None] * NCC
    for c in range(2):
        gets[c] = pltpu.async_copy(
            yw_hbm.at[cidx.at[pl.ds(c * CT * K, CT * K)]], rows[c % 2], semg[c % 2])
    for c in range(NCC):
        p = c % 2
        gets[c].wait()
        if c >= 2:
            puts[c - 2].wait()

        def row_body(r, _2, _p=p):
            for q in range(D // 16):
                sl = pl.ds(q * 16, 16)
                outv[_p][r, sl] = rows[_p][2 * r, sl] + rows[_p][2 * r + 1, sl]
            return 0

        lax.fori_loop(0, CT, row_body, 0)
        if c + 2 < NCC:
            gets[c + 2] = pltpu.async_copy(
                yw_hbm.at[cidx.at[pl.ds((c + 2) * CT * K, CT * K)]], rows[p], semg[p])
        puts[c] = pltpu.async_copy(outv[p], out_hbm.at[pl.ds(tbase + c * CT, CT)], semp[p])
    puts[NCC - 2].wait()
    puts[NCC - 1].wait()


@functools.lru_cache(maxsize=None)
def _combine_kernel():
    return pl.kernel(
        _combine_body,
        out_type=jax.ShapeDtypeStruct((T, D), jnp.float32),
        mesh=plsc.VectorSubcoreMesh(core_axis_name="c", subcore_axis_name="s",
                                    num_cores=NC, num_subcores=NS),
        compiler_params=pltpu.CompilerParams(needs_layout_passes=False),
        scratch_types=[
            pltpu.VMEM((TOK_W * K,), jnp.int32),
            [pltpu.VMEM((2 * CT, D), jnp.float32)] * 2,
            [pltpu.VMEM((CT, D), jnp.float32)] * 2,
            [pltpu.SemaphoreType.DMA] * 2,
            [pltpu.SemaphoreType.DMA] * 2,
        ],
    )


def kernel(hidden_states, router_w, router_b, w1, w2):
    flat = hidden_states.reshape(T, D)
    comb, g, rowmask = _router_call(flat, router_w, router_b.reshape(1, E))
    buf, gw = _dispatch_kernel()(flat, comb.reshape(TK), g.reshape(TK))
    yw = _ffn_call(buf.reshape(E, C, D), w1, w2, gw.reshape(E, C, 1),
                   rowmask.reshape(E, C, 1))
    out = _combine_kernel()(yw.reshape(EC, D), comb.reshape(TK))
    return out.reshape(B, S, D)
